# baseline (speedup 1.0000x reference)
import functools
import math

import jax
import jax.numpy as jnp
from jax import lax
from jax.experimental import pallas as pl
from jax.experimental.pallas import tpu as pltpu
from jax.experimental.pallas import tpu_sc as plsc

N = 10000
M = 32
ATOM = 128
NBR = 16
TILES = 32
EPT_RAW = N * M // TILES
CHUNK = 128
CH = (EPT_RAW + CHUNK - 1) // CHUNK
CH = CH + (-CH) % 8
EPT = CH * CHUNK
NPAD = N + 16


def _edge_scalar_body(x_ref, w1_ref, b1_ref, w2_ref, b2_ref, idx_ref,
                      o_ref, oidx_ref):
    x = x_ref[...]
    z = jnp.dot(x, w1_ref[...], preferred_element_type=jnp.float32)
    z = z + b1_ref[...][None, :]
    sp = jnp.maximum(z, 0.0) + jnp.log(1.0 + jnp.exp(-jnp.abs(z)))
    w2c = w2_ref[...][:, 0]
    s = jnp.sum(sp * w2c[None, :], axis=1) + b2_ref[0]
    pad = jnp.zeros((EPT - EPT_RAW,), jnp.float32)
    o_ref[...] = jnp.concatenate([s, pad]).reshape(1, CH, CHUNK)
    ipad = N + lax.rem(
        lax.broadcasted_iota(jnp.int32, (EPT - EPT_RAW,), 0),
        jnp.int32(NPAD - N))
    oidx_ref[...] = jnp.concatenate(
        [idx_ref[0, 0, :], ipad]).reshape(1, CH, CHUNK)


def _edge_scalars(nbr2d, W1, b1, W2, b2, idx2d):
    return pl.pallas_call(
        _edge_scalar_body,
        grid=(TILES,),
        in_specs=[
            pl.BlockSpec((EPT_RAW, NBR), lambda w: (w, 0)),
            pl.BlockSpec((NBR, NBR), lambda w: (0, 0)),
            pl.BlockSpec((NBR,), lambda w: (0,)),
            pl.BlockSpec((NBR, 9), lambda w: (0, 0)),
            pl.BlockSpec((9,), lambda w: (0,)),
            pl.BlockSpec((1, 1, EPT_RAW), lambda w: (w, 0, 0)),
        ],
        out_specs=[
            pl.BlockSpec((1, CH, CHUNK), lambda w: (w, 0, 0)),
            pl.BlockSpec((1, CH, CHUNK), lambda w: (w, 0, 0)),
        ],
        out_shape=[
            jax.ShapeDtypeStruct((TILES, CH, CHUNK), jnp.float32),
            jax.ShapeDtypeStruct((TILES, CH, CHUNK), jnp.int32),
        ],
    )(nbr2d, W1, b1, W2, b2, idx2d)


def _segment_body(idx_hbm, s_hbm, z_hbm, sp_hbm, cp_hbm,
                  idx_v, s_v, ones_v, acc_s, acc_c):
    c = lax.axis_index("c")
    s = lax.axis_index("s")
    w = c * 16 + s

    pltpu.sync_copy(idx_hbm.at[w], idx_v)
    pltpu.sync_copy(s_hbm.at[w], s_v)
    for i in range(CHUNK // 16):
        ones_v[pl.ds(i * 16, 16)] = jnp.full((16,), 1.0, jnp.float32)

    @pl.when(s == 0)
    def _zero():
        pltpu.sync_copy(z_hbm, acc_s)
        pltpu.sync_copy(z_hbm, acc_c)

    plsc.subcore_barrier()

    def body(j, carry):
        pltpu.sync_copy(s_v.at[j], acc_s.at[idx_v.at[j]], add=True)
        pltpu.sync_copy(ones_v, acc_c.at[idx_v.at[j]], add=True)
        return carry

    lax.fori_loop(0, CH, body, 0)

    plsc.subcore_barrier()

    @pl.when(s == 0)
    def _out():
        pltpu.sync_copy(acc_s, sp_hbm.at[c])
        pltpu.sync_copy(acc_c, cp_hbm.at[c])


def _segment_sums(idxp, s_pad, zeros_np):
    mesh = plsc.VectorSubcoreMesh(core_axis_name="c", subcore_axis_name="s")
    f = functools.partial(
        pl.kernel,
        mesh=mesh,
        out_type=[
            jax.ShapeDtypeStruct((2, NPAD), jnp.float32),
            jax.ShapeDtypeStruct((2, NPAD), jnp.float32),
        ],
        scratch_types=[
            pltpu.VMEM((CH, CHUNK), jnp.int32),
            pltpu.VMEM((CH, CHUNK), jnp.float32),
            pltpu.VMEM((CHUNK,), jnp.float32),
            pltpu.VMEM_SHARED((NPAD,), jnp.float32),
            pltpu.VMEM_SHARED((NPAD,), jnp.float32),
        ],
    )(_segment_body)
    return f(idxp, s_pad, zeros_np)


def _scale_matmul_body(a_ref, w_ref, sp_ref, cp_ref, o_ref):
    S = sp_ref[0, :] + sp_ref[1, :]
    C = cp_ref[0, :] + cp_ref[1, :]
    scale = S / jnp.maximum(C, 1.0) * (1.0 / math.sqrt(float(ATOM)))
    acc = jnp.dot(a_ref[...], w_ref[...], preferred_element_type=jnp.float32)
    o_ref[...] = acc * scale[:, None]


def _scale_matmul(atom_fea, tp_w, Sp, Cp):
    B = 1024
    grid = (N + B - 1) // B
    return pl.pallas_call(
        _scale_matmul_body,
        grid=(grid,),
        in_specs=[
            pl.BlockSpec((B, ATOM), lambda g: (g, 0)),
            pl.BlockSpec((ATOM, ATOM), lambda g: (0, 0)),
            pl.BlockSpec((2, B), lambda g: (0, g)),
            pl.BlockSpec((2, B), lambda g: (0, g)),
        ],
        out_specs=pl.BlockSpec((B, ATOM), lambda g: (g, 0)),
        out_shape=jax.ShapeDtypeStruct((N, ATOM), jnp.float32),
    )(atom_fea, tp_w, Sp, Cp)


def kernel(atom_fea, nbr_fea, nbr_idx, pos, W1, b1, W2, b2, tp_w):
    n, m, nbr = nbr_fea.shape
    idx2d = nbr_idx.reshape(TILES, 1, EPT_RAW).astype(jnp.int32)
    s_pad, idxp = _edge_scalars(
        nbr_fea.reshape(n * m, nbr), W1, b1, W2, b2, idx2d)

    zeros_np = jnp.zeros((NPAD,), jnp.float32)
    Sp, Cp = _segment_sums(idxp, s_pad, zeros_np)
    return _scale_matmul(atom_fea, tp_w, Sp, Cp)


# speedup vs baseline: 3.6721x; 3.6721x over previous
import functools
import math

import jax
import jax.numpy as jnp
from jax import lax
from jax.experimental import pallas as pl
from jax.experimental.pallas import tpu as pltpu
from jax.experimental.pallas import tpu_sc as plsc

N = 10000
M = 32
ATOM = 128
NBR = 16
TILES = 32
CHUNK = 128
R_BLK = 80
EROWS = M * R_BLK
CH = EROWS // TILES
NPAD = N + 16


def _edge_scalar_body(x_ref, w1_ref, b1_ref, w2_ref, b2_ref, idx_ref,
                      o_ref, oidx_ref):
    x = x_ref[0]
    zt = lax.dot_general(w1_ref[...], x, (((0,), (0,)), ((), ())),
                         preferred_element_type=jnp.float32)
    zt = zt + b1_ref[...][:, None]
    sp = jnp.maximum(zt, 0.0) + jnp.log(1.0 + jnp.exp(-jnp.abs(zt)))
    w2c = w2_ref[...][:, 0]
    s = jnp.sum(sp * w2c[:, None], axis=0) + b2_ref[0]
    pad = jnp.zeros((R_BLK * CHUNK - N,), jnp.float32)
    o_ref[...] = jnp.concatenate([s, pad]).reshape(R_BLK, CHUNK)
    ipad = N + lax.rem(
        lax.broadcasted_iota(jnp.int32, (R_BLK * CHUNK - N,), 0),
        jnp.int32(NPAD - N))
    oidx_ref[...] = jnp.concatenate(
        [idx_ref[0, 0, :], ipad]).reshape(R_BLK, CHUNK)


def _edge_scalars(nbr_t, W1, b1, W2, b2, idx_t):
    return pl.pallas_call(
        _edge_scalar_body,
        grid=(M,),
        in_specs=[
            pl.BlockSpec((1, NBR, N), lambda w: (w, 0, 0)),
            pl.BlockSpec((NBR, NBR), lambda w: (0, 0)),
            pl.BlockSpec((NBR,), lambda w: (0,)),
            pl.BlockSpec((NBR, 9), lambda w: (0, 0)),
            pl.BlockSpec((9,), lambda w: (0,)),
            pl.BlockSpec((1, 1, N), lambda w: (w, 0, 0)),
        ],
        out_specs=[
            pl.BlockSpec((R_BLK, CHUNK), lambda w: (w, 0)),
            pl.BlockSpec((R_BLK, CHUNK), lambda w: (w, 0)),
        ],
        out_shape=[
            jax.ShapeDtypeStruct((EROWS, CHUNK), jnp.float32),
            jax.ShapeDtypeStruct((EROWS, CHUNK), jnp.int32),
        ],
    )(nbr_t, W1, b1, W2, b2, idx_t)


def _segment_body(idx_hbm, s_hbm, z_hbm, sp_hbm, cp_hbm,
                  idx_v, s_v, ones_v, acc_s, acc_c):
    c = lax.axis_index("c")
    s = lax.axis_index("s")
    w = c * 16 + s

    pltpu.sync_copy(idx_hbm.at[pl.ds(w * CH, CH)], idx_v)
    pltpu.sync_copy(s_hbm.at[pl.ds(w * CH, CH)], s_v)
    for i in range(CHUNK // 16):
        ones_v[pl.ds(i * 16, 16)] = jnp.full((16,), 1.0, jnp.float32)

    @pl.when(s == 0)
    def _zero():
        pltpu.sync_copy(z_hbm, acc_s)
        pltpu.sync_copy(z_hbm, acc_c)

    plsc.subcore_barrier()

    def body(j, carry):
        pltpu.sync_copy(s_v.at[j], acc_s.at[idx_v.at[j]], add=True)
        pltpu.sync_copy(ones_v, acc_c.at[idx_v.at[j]], add=True)
        return carry

    lax.fori_loop(0, CH, body, 0)

    plsc.subcore_barrier()

    @pl.when(s == 0)
    def _out():
        pltpu.sync_copy(acc_s, sp_hbm.at[c])
        pltpu.sync_copy(acc_c, cp_hbm.at[c])


def _segment_sums(idxp, s_pad, zeros_np):
    mesh = plsc.VectorSubcoreMesh(core_axis_name="c", subcore_axis_name="s")
    f = functools.partial(
        pl.kernel,
        mesh=mesh,
        out_type=[
            jax.ShapeDtypeStruct((2, NPAD), jnp.float32),
            jax.ShapeDtypeStruct((2, NPAD), jnp.float32),
        ],
        scratch_types=[
            pltpu.VMEM((CH, CHUNK), jnp.int32),
            pltpu.VMEM((CH, CHUNK), jnp.float32),
            pltpu.VMEM((CHUNK,), jnp.float32),
            pltpu.VMEM_SHARED((NPAD,), jnp.float32),
            pltpu.VMEM_SHARED((NPAD,), jnp.float32),
        ],
    )(_segment_body)
    return f(idxp, s_pad, zeros_np)


def _scale_matmul_body(a_ref, w_ref, sp_ref, cp_ref, o_ref):
    S = sp_ref[0, :] + sp_ref[1, :]
    C = cp_ref[0, :] + cp_ref[1, :]
    scale = S / jnp.maximum(C, 1.0) * (1.0 / math.sqrt(float(ATOM)))
    acc = jnp.dot(a_ref[...], w_ref[...], preferred_element_type=jnp.float32)
    o_ref[...] = acc * scale[:, None]


def _scale_matmul(atom_fea, tp_w, Sp, Cp):
    B = 1024
    grid = (N + B - 1) // B
    return pl.pallas_call(
        _scale_matmul_body,
        grid=(grid,),
        in_specs=[
            pl.BlockSpec((B, ATOM), lambda g: (g, 0)),
            pl.BlockSpec((ATOM, ATOM), lambda g: (0, 0)),
            pl.BlockSpec((2, B), lambda g: (0, g)),
            pl.BlockSpec((2, B), lambda g: (0, g)),
        ],
        out_specs=pl.BlockSpec((B, ATOM), lambda g: (g, 0)),
        out_shape=jax.ShapeDtypeStruct((N, ATOM), jnp.float32),
    )(atom_fea, tp_w, Sp, Cp)


def kernel(atom_fea, nbr_fea, nbr_idx, pos, W1, b1, W2, b2, tp_w):
    nbr_t = jnp.transpose(nbr_fea, (1, 2, 0))
    idx_t = jnp.transpose(nbr_idx, (1, 0)).reshape(M, 1, N).astype(jnp.int32)
    s_pad, idxp = _edge_scalars(nbr_t, W1, b1, W2, b2, idx_t)

    zeros_np = jnp.zeros((NPAD,), jnp.float32)
    Sp, Cp = _segment_sums(idxp, s_pad, zeros_np)
    return _scale_matmul(atom_fea, tp_w, Sp, Cp)


# speedup vs baseline: 4.1550x; 1.1315x over previous
import functools
import math

import jax
import jax.numpy as jnp
from jax import lax
from jax.experimental import pallas as pl
from jax.experimental.pallas import tpu as pltpu
from jax.experimental.pallas import tpu_sc as plsc

N = 10000
M = 32
ATOM = 128
NBR = 16
TILES = 32
CHUNK = 128
R_BLK = 80
EROWS = M * R_BLK
CH = EROWS // TILES
NPAD = N + 16


def _edge_scalar_body(x_ref, w1_ref, b1_ref, w2_ref, b2_ref, idx_ref,
                      o_ref, oidx_ref):
    x = x_ref[0]
    zt = lax.dot_general(w1_ref[...], x, (((0,), (0,)), ((), ())),
                         preferred_element_type=jnp.float32)
    zt = zt + b1_ref[...][:, None]
    sp = jnp.maximum(zt, 0.0) + jnp.log(1.0 + jnp.exp(-jnp.abs(zt)))
    w2c = w2_ref[...][:, 0]
    s = jnp.sum(sp * w2c[:, None], axis=0) + b2_ref[0]
    pad = jnp.zeros((R_BLK * CHUNK - N,), jnp.float32)
    o_ref[...] = jnp.concatenate([s, pad]).reshape(R_BLK, CHUNK)
    ipad = N + lax.rem(
        lax.broadcasted_iota(jnp.int32, (R_BLK * CHUNK - N,), 0),
        jnp.int32(NPAD - N))
    oidx_ref[...] = jnp.concatenate(
        [idx_ref[0, 0, :], ipad]).reshape(R_BLK, CHUNK)


def _edge_scalars(nbr_t, W1, b1, W2, b2, idx_t):
    return pl.pallas_call(
        _edge_scalar_body,
        grid=(M,),
        in_specs=[
            pl.BlockSpec((1, NBR, N), lambda w: (w, 0, 0)),
            pl.BlockSpec((NBR, NBR), lambda w: (0, 0)),
            pl.BlockSpec((NBR,), lambda w: (0,)),
            pl.BlockSpec((NBR, 9), lambda w: (0, 0)),
            pl.BlockSpec((9,), lambda w: (0,)),
            pl.BlockSpec((1, 1, N), lambda w: (w, 0, 0)),
        ],
        out_specs=[
            pl.BlockSpec((R_BLK, CHUNK), lambda w: (w, 0)),
            pl.BlockSpec((R_BLK, CHUNK), lambda w: (w, 0)),
        ],
        out_shape=[
            jax.ShapeDtypeStruct((EROWS, CHUNK), jnp.float32),
            jax.ShapeDtypeStruct((EROWS, CHUNK), jnp.int32),
        ],
    )(nbr_t, W1, b1, W2, b2, idx_t)


def _segment_body(idx_hbm, s_hbm, z_hbm, sp_hbm, cp_hbm,
                  idx_v, s_v, ones_v, acc_s, acc_c, sem):
    c = lax.axis_index("c")
    s = lax.axis_index("s")
    w = c * 16 + s

    pltpu.sync_copy(idx_hbm.at[pl.ds(w * CH, CH)], idx_v)
    pltpu.sync_copy(s_hbm.at[pl.ds(w * CH, CH)], s_v)
    for i in range(CHUNK // 16):
        ones_v[pl.ds(i * 16, 16)] = jnp.full((16,), 1.0, jnp.float32)

    @pl.when(s == 0)
    def _zero():
        pltpu.sync_copy(z_hbm, acc_s)
        pltpu.sync_copy(z_hbm, acc_c)

    plsc.subcore_barrier()

    GRP = 8

    def body(g, carry):
        copies = []
        for b in range(GRP):
            j = g * GRP + b
            copies.append(pltpu.async_copy(
                s_v.at[j], acc_s.at[idx_v.at[j]], sem, add=True))
            copies.append(pltpu.async_copy(
                ones_v, acc_c.at[idx_v.at[j]], sem, add=True))
        for cp in copies:
            cp.wait()
        return carry

    lax.fori_loop(0, CH // GRP, body, 0)

    plsc.subcore_barrier()

    @pl.when(s == 0)
    def _out():
        pltpu.sync_copy(acc_s, sp_hbm.at[c])
        pltpu.sync_copy(acc_c, cp_hbm.at[c])


def _segment_sums(idxp, s_pad, zeros_np):
    mesh = plsc.VectorSubcoreMesh(core_axis_name="c", subcore_axis_name="s")
    f = functools.partial(
        pl.kernel,
        mesh=mesh,
        out_type=[
            jax.ShapeDtypeStruct((2, NPAD), jnp.float32),
            jax.ShapeDtypeStruct((2, NPAD), jnp.float32),
        ],
        scratch_types=[
            pltpu.VMEM((CH, CHUNK), jnp.int32),
            pltpu.VMEM((CH, CHUNK), jnp.float32),
            pltpu.VMEM((CHUNK,), jnp.float32),
            pltpu.VMEM_SHARED((NPAD,), jnp.float32),
            pltpu.VMEM_SHARED((NPAD,), jnp.float32),
            pltpu.SemaphoreType.DMA,
        ],
    )(_segment_body)
    return f(idxp, s_pad, zeros_np)


def _scale_matmul_body(a_ref, w_ref, sp_ref, cp_ref, o_ref):
    S = sp_ref[0, :] + sp_ref[1, :]
    C = cp_ref[0, :] + cp_ref[1, :]
    scale = S / jnp.maximum(C, 1.0) * (1.0 / math.sqrt(float(ATOM)))
    acc = jnp.dot(a_ref[...], w_ref[...], preferred_element_type=jnp.float32)
    o_ref[...] = acc * scale[:, None]


def _scale_matmul(atom_fea, tp_w, Sp, Cp):
    B = 1024
    grid = (N + B - 1) // B
    return pl.pallas_call(
        _scale_matmul_body,
        grid=(grid,),
        in_specs=[
            pl.BlockSpec((B, ATOM), lambda g: (g, 0)),
            pl.BlockSpec((ATOM, ATOM), lambda g: (0, 0)),
            pl.BlockSpec((2, B), lambda g: (0, g)),
            pl.BlockSpec((2, B), lambda g: (0, g)),
        ],
        out_specs=pl.BlockSpec((B, ATOM), lambda g: (g, 0)),
        out_shape=jax.ShapeDtypeStruct((N, ATOM), jnp.float32),
    )(atom_fea, tp_w, Sp, Cp)


def kernel(atom_fea, nbr_fea, nbr_idx, pos, W1, b1, W2, b2, tp_w):
    nbr_t = jnp.transpose(nbr_fea, (1, 2, 0))
    idx_t = jnp.transpose(nbr_idx, (1, 0)).reshape(M, 1, N).astype(jnp.int32)
    s_pad, idxp = _edge_scalars(nbr_t, W1, b1, W2, b2, idx_t)

    zeros_np = jnp.zeros((NPAD,), jnp.float32)
    Sp, Cp = _segment_sums(idxp, s_pad, zeros_np)
    return _scale_matmul(atom_fea, tp_w, Sp, Cp)


# speedup vs baseline: 5.0583x; 1.2174x over previous
import functools
import math

import jax
import jax.numpy as jnp
from jax import lax
from jax.experimental import pallas as pl
from jax.experimental.pallas import tpu as pltpu
from jax.experimental.pallas import tpu_sc as plsc

N = 10000
M = 32
ATOM = 128
NBR = 16
TILES = 32
CHUNK = 128
R_BLK = 80
EROWS = M * R_BLK
CH = EROWS // TILES
NPAD = N + 16


KB = 4


def _edge_scalar_body(x_ref, w1_ref, b1_ref, w2_ref, b2_ref, idx_ref,
                      o_ref, oidx_ref):
    w2c = w2_ref[...][:, 0]
    pad = jnp.zeros((R_BLK * CHUNK - N,), jnp.float32)
    ipad = N + lax.rem(
        lax.broadcasted_iota(jnp.int32, (R_BLK * CHUNK - N,), 0),
        jnp.int32(NPAD - N))
    s_parts, i_parts = [], []
    for kk in range(KB):
        x = x_ref[kk]
        zt = lax.dot_general(w1_ref[...], x, (((0,), (0,)), ((), ())),
                             preferred_element_type=jnp.float32)
        zt = zt + b1_ref[...][:, None]
        sp = jnp.maximum(zt, 0.0) + jnp.log(1.0 + jnp.exp(-jnp.abs(zt)))
        s = jnp.sum(sp * w2c[:, None], axis=0) + b2_ref[0]
        s_parts += [s, pad]
        i_parts += [idx_ref[kk, 0, :], ipad]
    o_ref[...] = jnp.concatenate(s_parts).reshape(KB * R_BLK, CHUNK)
    oidx_ref[...] = jnp.concatenate(i_parts).reshape(KB * R_BLK, CHUNK)


def _edge_scalars(nbr_t, W1, b1, W2, b2, idx_t):
    return pl.pallas_call(
        _edge_scalar_body,
        grid=(M // KB,),
        in_specs=[
            pl.BlockSpec((KB, NBR, N), lambda w: (w, 0, 0)),
            pl.BlockSpec((NBR, NBR), lambda w: (0, 0)),
            pl.BlockSpec((NBR,), lambda w: (0,)),
            pl.BlockSpec((NBR, 9), lambda w: (0, 0)),
            pl.BlockSpec((9,), lambda w: (0,)),
            pl.BlockSpec((KB, 1, N), lambda w: (w, 0, 0)),
        ],
        out_specs=[
            pl.BlockSpec((KB * R_BLK, CHUNK), lambda w: (w, 0)),
            pl.BlockSpec((KB * R_BLK, CHUNK), lambda w: (w, 0)),
        ],
        out_shape=[
            jax.ShapeDtypeStruct((EROWS, CHUNK), jnp.float32),
            jax.ShapeDtypeStruct((EROWS, CHUNK), jnp.int32),
        ],
    )(nbr_t, W1, b1, W2, b2, idx_t)


def _segment_body(idx_hbm, s_hbm, z_hbm, sp_hbm, cp_hbm,
                  idx_v, s_v, ones_v, acc_s, acc_c, sem):
    c = lax.axis_index("c")
    s = lax.axis_index("s")
    w = c * 16 + s

    pltpu.sync_copy(idx_hbm.at[pl.ds(w * CH, CH)], idx_v)
    pltpu.sync_copy(s_hbm.at[pl.ds(w * CH, CH)], s_v)
    for i in range(CHUNK // 16):
        ones_v[pl.ds(i * 16, 16)] = jnp.full((16,), 1.0, jnp.float32)

    @pl.when(s == 0)
    def _zero():
        pltpu.sync_copy(z_hbm, acc_s)
        pltpu.sync_copy(z_hbm, acc_c)

    plsc.subcore_barrier()

    GRP = 8

    def body(g, carry):
        copies = []
        for b in range(GRP):
            j = g * GRP + b
            copies.append(pltpu.async_copy(
                s_v.at[j], acc_s.at[idx_v.at[j]], sem, add=True))
            copies.append(pltpu.async_copy(
                ones_v, acc_c.at[idx_v.at[j]], sem, add=True))
        for cp in copies:
            cp.wait()
        return carry

    lax.fori_loop(0, CH // GRP, body, 0)

    plsc.subcore_barrier()

    @pl.when(s == 0)
    def _out():
        pltpu.sync_copy(acc_s, sp_hbm.at[c])
        pltpu.sync_copy(acc_c, cp_hbm.at[c])


def _segment_sums(idxp, s_pad, zeros_np):
    mesh = plsc.VectorSubcoreMesh(core_axis_name="c", subcore_axis_name="s")
    f = functools.partial(
        pl.kernel,
        mesh=mesh,
        out_type=[
            jax.ShapeDtypeStruct((2, NPAD), jnp.float32),
            jax.ShapeDtypeStruct((2, NPAD), jnp.float32),
        ],
        scratch_types=[
            pltpu.VMEM((CH, CHUNK), jnp.int32),
            pltpu.VMEM((CH, CHUNK), jnp.float32),
            pltpu.VMEM((CHUNK,), jnp.float32),
            pltpu.VMEM_SHARED((NPAD,), jnp.float32),
            pltpu.VMEM_SHARED((NPAD,), jnp.float32),
            pltpu.SemaphoreType.DMA,
        ],
    )(_segment_body)
    return f(idxp, s_pad, zeros_np)


def _scale_matmul_body(a_ref, w_ref, sp_ref, cp_ref, o_ref):
    S = sp_ref[0, :] + sp_ref[1, :]
    C = cp_ref[0, :] + cp_ref[1, :]
    scale = S / jnp.maximum(C, 1.0) * (1.0 / math.sqrt(float(ATOM)))
    acc = jnp.dot(a_ref[...], w_ref[...], preferred_element_type=jnp.float32)
    o_ref[...] = acc * scale[:, None]


def _scale_matmul(atom_fea, tp_w, Sp, Cp):
    B = 1024
    grid = (N + B - 1) // B
    return pl.pallas_call(
        _scale_matmul_body,
        grid=(grid,),
        in_specs=[
            pl.BlockSpec((B, ATOM), lambda g: (g, 0)),
            pl.BlockSpec((ATOM, ATOM), lambda g: (0, 0)),
            pl.BlockSpec((2, B), lambda g: (0, g)),
            pl.BlockSpec((2, B), lambda g: (0, g)),
        ],
        out_specs=pl.BlockSpec((B, ATOM), lambda g: (g, 0)),
        out_shape=jax.ShapeDtypeStruct((N, ATOM), jnp.float32),
    )(atom_fea, tp_w, Sp, Cp)


def kernel(atom_fea, nbr_fea, nbr_idx, pos, W1, b1, W2, b2, tp_w):
    nbr_t = jnp.transpose(nbr_fea, (1, 2, 0))
    idx_t = jnp.transpose(nbr_idx, (1, 0)).reshape(M, 1, N).astype(jnp.int32)
    s_pad, idxp = _edge_scalars(nbr_t, W1, b1, W2, b2, idx_t)

    zeros_np = jnp.zeros((NPAD,), jnp.float32)
    Sp, Cp = _segment_sums(idxp, s_pad, zeros_np)
    return _scale_matmul(atom_fea, tp_w, Sp, Cp)
